# baseline (speedup 1.0000x reference)
import functools

import jax
import jax.numpy as jnp
from jax import lax
from jax.experimental import pallas as pl
from jax.experimental.pallas import tpu as pltpu
from jax.experimental.pallas import tpu_sc as plsc

_FEATURES = ["restaurants", "user_rating", "artgallery", "beaches", "mall",
             "museums", "parks", "pubs_bars", "resorts", "theatres", "zoo"]
_NF = len(_FEATURES)
_BATCH = 16384
_EMB = 32
_NB = 100
_LANES = 128
_BLK = 2048
_NC, _NS = 2, 16
_NW = _NC * _NS
_BPW = _BATCH // _NW
_NCH = 4
_CH = _BPW // _NCH
_WIDTH = _EMB + _NF * (_EMB + 1)


def _sc_gather_body(nusers, uid_hbm, table_hbm, out_hbm, idx_v, rows_v, sem):
    wid = lax.axis_index("s") * _NC + lax.axis_index("c")
    pltpu.sync_copy(uid_hbm.at[wid], idx_v)
    for j in range(_NCH):
        for i in range(_CH // 16):
            u = idx_v[j, pl.ds(i * 16, 16)]
            ok = (u >= 0) & (u < nusers)
            idx_v[j, pl.ds(i * 16, 16)] = jnp.where(ok, u + 1, 0)
    copies = []
    for j in range(_NCH):
        copies.append(pltpu.async_copy(
            table_hbm.at[idx_v.at[j]],
            rows_v.at[pl.ds(j * _CH, _CH)], sem))
    for c in copies:
        c.wait()
    pltpu.sync_copy(rows_v, out_hbm.at[pl.ds(wid * _BPW, _BPW)])


def _sc_gather(user_id, user_table):
    nusers = user_table.shape[0] - 1
    mesh = plsc.VectorSubcoreMesh(core_axis_name="c", subcore_axis_name="s")
    fn = pl.kernel(
        functools.partial(_sc_gather_body, nusers),
        mesh=mesh,
        compiler_params=pltpu.CompilerParams(use_tc_tiling_on_sc=False),
        out_type=jax.ShapeDtypeStruct((_BATCH, _EMB), jnp.float32),
        scratch_types=[
            pltpu.VMEM((_NCH, _CH), jnp.int32),
            pltpu.VMEM((_BPW, _EMB), jnp.float32),
            pltpu.SemaphoreType.DMA,
        ],
    )
    return fn(user_id.reshape(_NW, _NCH, _CH), user_table)


def _tc_body(user_ref, x_ref, tpad_ref, lo_ref, hi_ref, ab_ref, out_ref):
    xv = x_ref[...]
    norms = xv * ab_ref[0:1, :] + ab_ref[1:2, :]
    ones_row = jnp.ones((1, _LANES), jnp.float32)
    parts = [user_ref[...]]
    for f in range(_NF):
        xb = jnp.dot(xv[:, f:f + 1], ones_row,
                     preferred_element_type=jnp.float32,
                     precision=lax.Precision.HIGHEST)
        oh = ((xb >= lo_ref[f:f + 1, :]) &
              (xb < hi_ref[f:f + 1, :])).astype(jnp.float32)
        emb = jnp.dot(oh, tpad_ref[f], preferred_element_type=jnp.float32)
        parts.append(emb)
        parts.append(norms[:, f:f + 1])
    out_ref[...] = jnp.concatenate(parts, axis=1)


def kernel(user_id, user_vocab, user_table,
           restaurants, restaurants_table, restaurants_bnd, restaurants_mean, restaurants_var,
           user_rating, user_rating_table, user_rating_bnd, user_rating_mean, user_rating_var,
           artgallery, artgallery_table, artgallery_bnd, artgallery_mean, artgallery_var,
           beaches, beaches_table, beaches_bnd, beaches_mean, beaches_var,
           mall, mall_table, mall_bnd, mall_mean, mall_var,
           museums, museums_table, museums_bnd, museums_mean, museums_var,
           parks, parks_table, parks_bnd, parks_mean, parks_var,
           pubs_bars, pubs_bars_table, pubs_bars_bnd, pubs_bars_mean, pubs_bars_var,
           resorts, resorts_table, resorts_bnd, resorts_mean, resorts_var,
           theatres, theatres_table, theatres_bnd, theatres_mean, theatres_var,
           zoo, zoo_table, zoo_bnd, zoo_mean, zoo_var):
    env = dict(locals())
    xs = [env[n] for n in _FEATURES]
    tables = [env[n + "_table"] for n in _FEATURES]
    bnds = [env[n + "_bnd"] for n in _FEATURES]
    means = [env[n + "_mean"] for n in _FEATURES]
    variances = [env[n + "_var"] for n in _FEATURES]

    user_rows = _sc_gather(user_id, user_table)

    X = jnp.stack(xs, axis=1)
    tpad = jnp.stack([jnp.pad(t, ((0, _LANES - t.shape[0]), (0, 0)))
                      for t in tables])
    inf = jnp.full((_LANES - _NB - 1,), jnp.inf, jnp.float32)
    lo = jnp.stack([jnp.concatenate([jnp.array([-jnp.inf], jnp.float32), b, inf])
                    for b in bnds])
    hi = jnp.stack([jnp.concatenate([b, jnp.full((_LANES - _NB,), jnp.inf, jnp.float32)])
                    for b in bnds])
    a = 1.0 / jnp.sqrt(jnp.stack(variances))
    ab = jnp.stack([a, -jnp.stack(means) * a])

    grid = (_BATCH // _BLK,)
    return pl.pallas_call(
        _tc_body,
        grid=grid,
        in_specs=[
            pl.BlockSpec((_BLK, _EMB), lambda i: (i, 0)),
            pl.BlockSpec((_BLK, _NF), lambda i: (i, 0)),
            pl.BlockSpec((_NF, _LANES, _EMB), lambda i: (0, 0, 0)),
            pl.BlockSpec((_NF, _LANES), lambda i: (0, 0)),
            pl.BlockSpec((_NF, _LANES), lambda i: (0, 0)),
            pl.BlockSpec((2, _NF), lambda i: (0, 0)),
        ],
        out_specs=pl.BlockSpec((_BLK, _WIDTH), lambda i: (i, 0)),
        out_shape=jax.ShapeDtypeStruct((_BATCH, _WIDTH), jnp.float32),
    )(user_rows, X, tpad, lo, hi, ab)


# speedup vs baseline: 1.5032x; 1.5032x over previous
import functools

import jax
import jax.numpy as jnp
from jax import lax
from jax.experimental import pallas as pl
from jax.experimental.pallas import tpu as pltpu
from jax.experimental.pallas import tpu_sc as plsc

_FEATURES = ["restaurants", "user_rating", "artgallery", "beaches", "mall",
             "museums", "parks", "pubs_bars", "resorts", "theatres", "zoo"]
_NF = len(_FEATURES)
_BATCH = 16384
_EMB = 32
_NB = 100
_LANES = 128
_BLK = 2048
_NC, _NS = 2, 16
_NW = _NC * _NS
_BPW = _BATCH // _NW
_NCH = 4
_CH = _BPW // _NCH
_WIDTH = _EMB + _NF * (_EMB + 1)


def _sc_gather_body(nusers, uid_hbm, table_hbm, out_hbm, idx_v, rows_v, sem):
    wid = lax.axis_index("s") * _NC + lax.axis_index("c")
    pltpu.sync_copy(uid_hbm.at[wid], idx_v)
    for j in range(_NCH):
        for i in range(_CH // 16):
            u = idx_v[j, pl.ds(i * 16, 16)]
            ok = (u >= 0) & (u < nusers)
            idx_v[j, pl.ds(i * 16, 16)] = jnp.where(ok, u + 1, 0)
    copies = []
    for j in range(_NCH):
        copies.append(pltpu.async_copy(
            table_hbm.at[idx_v.at[j]],
            rows_v.at[pl.ds(j * _CH, _CH)], sem))
    for c in copies:
        c.wait()
    pltpu.sync_copy(rows_v, out_hbm.at[pl.ds(wid * _BPW, _BPW)])


def _sc_gather(user_id, user_table):
    nusers = user_table.shape[0] - 1
    mesh = plsc.VectorSubcoreMesh(core_axis_name="c", subcore_axis_name="s")
    fn = pl.kernel(
        functools.partial(_sc_gather_body, nusers),
        mesh=mesh,
        compiler_params=pltpu.CompilerParams(use_tc_tiling_on_sc=False),
        out_type=jax.ShapeDtypeStruct((_BATCH, _EMB), jnp.float32),
        scratch_types=[
            pltpu.VMEM((_NCH, _CH), jnp.int32),
            pltpu.VMEM((_BPW, _EMB), jnp.float32),
            pltpu.SemaphoreType.DMA,
        ],
    )
    return fn(user_id.reshape(_NW, _NCH, _CH), user_table)


def _tc_body(user_ref, x_ref, tpad_ref, lo_ref, hi_ref, ab_ref, out_ref):
    xv = x_ref[...]
    norms = xv * ab_ref[0:1, :] + ab_ref[1:2, :]
    parts = [user_ref[...]]
    for f in range(_NF):
        xc = xv[:, f:f + 1]
        oh = ((xc >= lo_ref[f:f + 1, :]) &
              (xc < hi_ref[f:f + 1, :])).astype(jnp.float32)
        emb = jnp.dot(oh, tpad_ref[f], preferred_element_type=jnp.float32)
        parts.append(emb)
        parts.append(norms[:, f:f + 1])
    out_ref[...] = jnp.concatenate(parts, axis=1)


def kernel(user_id, user_vocab, user_table,
           restaurants, restaurants_table, restaurants_bnd, restaurants_mean, restaurants_var,
           user_rating, user_rating_table, user_rating_bnd, user_rating_mean, user_rating_var,
           artgallery, artgallery_table, artgallery_bnd, artgallery_mean, artgallery_var,
           beaches, beaches_table, beaches_bnd, beaches_mean, beaches_var,
           mall, mall_table, mall_bnd, mall_mean, mall_var,
           museums, museums_table, museums_bnd, museums_mean, museums_var,
           parks, parks_table, parks_bnd, parks_mean, parks_var,
           pubs_bars, pubs_bars_table, pubs_bars_bnd, pubs_bars_mean, pubs_bars_var,
           resorts, resorts_table, resorts_bnd, resorts_mean, resorts_var,
           theatres, theatres_table, theatres_bnd, theatres_mean, theatres_var,
           zoo, zoo_table, zoo_bnd, zoo_mean, zoo_var):
    env = dict(locals())
    xs = [env[n] for n in _FEATURES]
    tables = [env[n + "_table"] for n in _FEATURES]
    bnds = [env[n + "_bnd"] for n in _FEATURES]
    means = [env[n + "_mean"] for n in _FEATURES]
    variances = [env[n + "_var"] for n in _FEATURES]

    user_rows = _sc_gather(user_id, user_table)

    X = jnp.stack(xs, axis=1)
    tpad = jnp.stack([jnp.pad(t, ((0, _LANES - t.shape[0]), (0, 0)))
                      for t in tables])
    inf = jnp.full((_LANES - _NB - 1,), jnp.inf, jnp.float32)
    lo = jnp.stack([jnp.concatenate([jnp.array([-jnp.inf], jnp.float32), b, inf])
                    for b in bnds])
    hi = jnp.stack([jnp.concatenate([b, jnp.full((_LANES - _NB,), jnp.inf, jnp.float32)])
                    for b in bnds])
    a = 1.0 / jnp.sqrt(jnp.stack(variances))
    ab = jnp.stack([a, -jnp.stack(means) * a])

    grid = (_BATCH // _BLK,)
    return pl.pallas_call(
        _tc_body,
        grid=grid,
        in_specs=[
            pl.BlockSpec((_BLK, _EMB), lambda i: (i, 0)),
            pl.BlockSpec((_BLK, _NF), lambda i: (i, 0)),
            pl.BlockSpec((_NF, _LANES, _EMB), lambda i: (0, 0, 0)),
            pl.BlockSpec((_NF, _LANES), lambda i: (0, 0)),
            pl.BlockSpec((_NF, _LANES), lambda i: (0, 0)),
            pl.BlockSpec((2, _NF), lambda i: (0, 0)),
        ],
        out_specs=pl.BlockSpec((_BLK, _WIDTH), lambda i: (i, 0)),
        out_shape=jax.ShapeDtypeStruct((_BATCH, _WIDTH), jnp.float32),
    )(user_rows, X, tpad, lo, hi, ab)
